# baseline (speedup 1.0000x reference)
import functools

import jax
import jax.numpy as jnp
from jax.experimental import pallas as pl
from jax.experimental.pallas import tpu as pltpu

FB_CHUNKS = 8
IDX_BLK = 16384


def _fb_copies(f_ref, fb_ref, out_ref, sem):
    n = f_ref.shape[0]
    s = out_ref.shape[0]
    rest = s - n
    copies = [pltpu.make_async_copy(f_ref, out_ref.at[pl.ds(0, n)], sem.at[0])]
    chunk = (rest // FB_CHUNKS) // 8 * 8
    off = 0
    for c in range(FB_CHUNKS):
        size = chunk if c < FB_CHUNKS - 1 else rest - chunk * (FB_CHUNKS - 1)
        copies.append(
            pltpu.make_async_copy(
                fb_ref.at[pl.ds(off, size)],
                out_ref.at[pl.ds(n + off, size)],
                sem.at[1 + c],
            )
        )
        off += size
    return copies


def _body(f_ref, idx_ref, fb_ref, idxb_ref, out_ref, idxo_ref, sem):
    i = pl.program_id(0)
    last = pl.num_programs(0) - 1

    @pl.when(i == 0)
    def _():
        for c in _fb_copies(f_ref, fb_ref, out_ref, sem):
            c.start()
        idxo_ref[...] = idx_ref[...]

    @pl.when(i > 0)
    def _():
        idxo_ref[...] = idxb_ref[...]

    @pl.when(i == last)
    def _():
        for c in _fb_copies(f_ref, fb_ref, out_ref, sem):
            c.wait()


def kernel(f, idx, fb, idx_bank):
    f2 = f.reshape(-1, f.shape[-1])
    idx2 = idx.reshape(-1)
    N, F = f2.shape
    S = fb.shape[0]
    assert N == IDX_BLK
    grid = (pl.cdiv(S, IDX_BLK),)

    out_fb, out_idx = pl.pallas_call(
        _body,
        grid=grid,
        in_specs=[
            pl.BlockSpec(memory_space=pl.ANY),
            pl.BlockSpec((IDX_BLK,), lambda i: (0,)),
            pl.BlockSpec(memory_space=pl.ANY),
            pl.BlockSpec((IDX_BLK,), lambda i: (jnp.maximum(i - 1, 0),)),
        ],
        out_specs=[
            pl.BlockSpec(memory_space=pl.ANY),
            pl.BlockSpec((IDX_BLK,), lambda i: (i,)),
        ],
        out_shape=[
            jax.ShapeDtypeStruct((S, F), fb.dtype),
            jax.ShapeDtypeStruct((S,), idx_bank.dtype),
        ],
        scratch_shapes=[pltpu.SemaphoreType.DMA((1 + FB_CHUNKS,))],
    )(f2, idx2, fb, idx_bank)

    return (out_fb, out_idx)


# speedup vs baseline: 15.9323x; 15.9323x over previous
import functools

import jax
import jax.numpy as jnp
from jax.experimental import pallas as pl
from jax.experimental.pallas import tpu as pltpu

BLK = 16384


def _copy_body(f_ref, idx_ref, fb_ref, idxb_ref, out_ref, idxo_ref, *, nf):
    i = pl.program_id(0)

    @pl.when(i < nf)
    def _():
        out_ref[...] = f_ref[...]
        idxo_ref[...] = idx_ref[...]

    @pl.when(i >= nf)
    def _():
        out_ref[...] = fb_ref[...]
        idxo_ref[...] = idxb_ref[...]


def kernel(f, idx, fb, idx_bank):
    f2 = f.reshape(-1, f.shape[-1])
    idx2 = idx.reshape(-1)
    N, F = f2.shape
    S = fb.shape[0]
    assert N % BLK == 0
    nf = N // BLK
    grid = (pl.cdiv(S, BLK),)

    body = functools.partial(_copy_body, nf=nf)

    out_fb, out_idx = pl.pallas_call(
        body,
        grid=grid,
        in_specs=[
            pl.BlockSpec((BLK, F), lambda i: (jnp.minimum(i, nf - 1), 0)),
            pl.BlockSpec((BLK,), lambda i: (jnp.minimum(i, nf - 1),)),
            pl.BlockSpec((BLK, F), lambda i: (jnp.maximum(i - nf, 0), 0)),
            pl.BlockSpec((BLK,), lambda i: (jnp.maximum(i - nf, 0),)),
        ],
        out_specs=[
            pl.BlockSpec((BLK, F), lambda i: (i, 0)),
            pl.BlockSpec((BLK,), lambda i: (i,)),
        ],
        out_shape=[
            jax.ShapeDtypeStruct((S, F), fb.dtype),
            jax.ShapeDtypeStruct((S,), idx_bank.dtype),
        ],
    )(f2, idx2, fb, idx_bank)

    return (out_fb, out_idx)


# speedup vs baseline: 16.0054x; 1.0046x over previous
import functools

import jax
import jax.numpy as jnp
from jax.experimental import pallas as pl
from jax.experimental.pallas import tpu as pltpu

CH = 8000
NBUF = 6
LOOK = 4
IDX_BLK = 16384


def _read(f_ref, fb_ref, buf, rsem, c, n):
    return pltpu.make_async_copy(
        fb_ref.at[pl.ds(c * CH - n, CH)], buf, rsem
    )


def _body(f_ref, idx_ref, fb_ref, idxb_ref, out_ref, idxo_ref,
          bufs, rsems, wsems, *, nc, nf_full, c_mix, mix_f_rows, nidx):
    i = pl.program_id(0)

    @pl.when(i == 0)
    def _():
        for c in range(LOOK):
            b = c % NBUF
            if c < nf_full:
                pltpu.make_async_copy(
                    f_ref.at[pl.ds(c * CH, CH)], bufs.at[b], rsems.at[b]
                ).start()
            elif c == c_mix:
                pltpu.make_async_copy(
                    f_ref.at[pl.ds(c * CH, mix_f_rows)],
                    bufs.at[b, pl.ds(0, mix_f_rows)],
                    rsems.at[b],
                ).start()
                pltpu.make_async_copy(
                    fb_ref.at[pl.ds(0, CH - mix_f_rows)],
                    bufs.at[b, pl.ds(mix_f_rows, CH - mix_f_rows)],
                    rsems.at[b],
                ).start()
            else:
                _read(f_ref, fb_ref, bufs.at[b], rsems.at[b],
                      c, nf_full * CH + mix_f_rows).start()

    j = i + LOOK
    bj = jax.lax.rem(j, NBUF)

    @pl.when(j < nc)
    def _():
        @pl.when(j >= NBUF)
        def _():
            pltpu.make_async_copy(
                fb_ref.at[pl.ds(0, CH)], out_ref.at[pl.ds(0, CH)],
                wsems.at[bj],
            ).wait()
        _read(f_ref, fb_ref, bufs.at[bj], rsems.at[bj],
              j, nf_full * CH + mix_f_rows).start()

    bi = jax.lax.rem(i, NBUF)
    n = nf_full * CH + mix_f_rows

    @pl.when(i != c_mix)
    def _():
        pltpu.make_async_copy(
            fb_ref.at[pl.ds(0, CH)], bufs.at[bi], rsems.at[bi]
        ).wait()

    @pl.when(i == c_mix)
    def _():
        pltpu.make_async_copy(
            f_ref.at[pl.ds(0, mix_f_rows)],
            bufs.at[bi, pl.ds(0, mix_f_rows)], rsems.at[bi],
        ).wait()
        pltpu.make_async_copy(
            fb_ref.at[pl.ds(0, CH - mix_f_rows)],
            bufs.at[bi, pl.ds(mix_f_rows, CH - mix_f_rows)], rsems.at[bi],
        ).wait()

    pltpu.make_async_copy(
        bufs.at[bi], out_ref.at[pl.ds(i * CH, CH)], wsems.at[bi]
    ).start()

    @pl.when(i == nc - 1)
    def _():
        for b in range(NBUF):
            pltpu.make_async_copy(
                fb_ref.at[pl.ds(0, CH)], out_ref.at[pl.ds(0, CH)],
                wsems.at[b],
            ).wait()

    @pl.when(i == 0)
    def _():
        idxo_ref[...] = idx_ref[...]

    @pl.when((i > 0) & (i < nidx))
    def _():
        idxo_ref[...] = idxb_ref[...]


def kernel(f, idx, fb, idx_bank):
    f2 = f.reshape(-1, f.shape[-1])
    idx2 = idx.reshape(-1)
    N, F = f2.shape
    S = fb.shape[0]
    assert S % CH == 0
    nc = S // CH
    nf_full = N // CH
    mix_f_rows = N - nf_full * CH
    c_mix = nf_full if mix_f_rows else -1
    assert c_mix < LOOK
    nidx = pl.cdiv(S, IDX_BLK)
    nidx_in = nidx - 1

    body = functools.partial(
        _body, nc=nc, nf_full=nf_full, c_mix=c_mix,
        mix_f_rows=mix_f_rows, nidx=nidx,
    )

    out_fb, out_idx = pl.pallas_call(
        body,
        grid=(nc,),
        in_specs=[
            pl.BlockSpec(memory_space=pl.ANY),
            pl.BlockSpec((IDX_BLK,), lambda i: (0,)),
            pl.BlockSpec(memory_space=pl.ANY),
            pl.BlockSpec((IDX_BLK,), lambda i: (jnp.clip(i - 1, 0, nidx_in - 1),)),
        ],
        out_specs=[
            pl.BlockSpec(memory_space=pl.ANY),
            pl.BlockSpec((IDX_BLK,), lambda i: (jnp.minimum(i, nidx - 1),)),
        ],
        out_shape=[
            jax.ShapeDtypeStruct((S, F), fb.dtype),
            jax.ShapeDtypeStruct((S,), idx_bank.dtype),
        ],
        scratch_shapes=[
            pltpu.VMEM((NBUF, CH, F), fb.dtype),
            pltpu.SemaphoreType.DMA((NBUF,)),
            pltpu.SemaphoreType.DMA((NBUF,)),
        ],
    )(f2, idx2, fb, idx_bank)

    return (out_fb, out_idx)
